# baseline (speedup 1.0000x reference)
import functools

import jax
import jax.numpy as jnp
from jax import lax
from jax.experimental import pallas as pl
from jax.experimental.pallas import tpu as pltpu
from jax.experimental.pallas import tpu_sc as plsc

_CHUNK = 16
_LANES = 16


def _sc_pooled(x1, idx, rows, hw, n_workers):
    n_idx = idx.shape[0]
    rows_w = rows // n_workers
    n_chunks = rows_w // _CHUNK
    n_pairs = n_chunks // 2
    chunk_elems = _CHUNK * hw
    mesh = plsc.VectorSubcoreMesh(core_axis_name="c", subcore_axis_name="s")

    @functools.partial(
        pl.kernel, mesh=mesh,
        out_type=jax.ShapeDtypeStruct((rows * _LANES,), jnp.float32),
        scratch_types=[
            pltpu.VMEM((chunk_elems,), jnp.float32),
            pltpu.VMEM((chunk_elems,), jnp.float32),
            pltpu.VMEM((rows_w * _LANES,), jnp.float32),
            pltpu.VMEM((n_idx,), jnp.int32),
            pltpu.SemaphoreType.DMA,
            pltpu.SemaphoreType.DMA,
        ],
        compiler_params=pltpu.CompilerParams(needs_layout_passes=False),
    )
    def k(x_hbm, idx_hbm, out_hbm, buf0, buf1, obuf, idx_v, sem0, sem1):
        wid = lax.axis_index("s") * 2 + lax.axis_index("c")
        webase = wid * rows_w * hw
        pltpu.sync_copy(idx_hbm, idx_v)
        idx_regs = [idx_v[pl.ds(q * _LANES, _LANES)] for q in range(n_idx // _LANES)]

        def start_in(chunk, buf, sem):
            pltpu.make_async_copy(
                x_hbm.at[pl.ds(webase + chunk * chunk_elems, chunk_elems)],
                buf, sem).start()

        def wait_in(chunk, buf, sem):
            pltpu.make_async_copy(
                x_hbm.at[pl.ds(webase + chunk * chunk_elems, chunk_elems)],
                buf, sem).wait()

        def consume(chunk, buf):
            lbase = chunk * _CHUNK * _LANES
            for r in range(_CHUNK):
                acc = plsc.load_gather(buf, [idx_regs[0] + r * hw])
                for q in range(1, len(idx_regs)):
                    acc = acc + plsc.load_gather(buf, [idx_regs[q] + r * hw])
                obuf[pl.ds(lbase + r * _LANES, _LANES)] = acc

        start_in(0, buf0, sem0)
        start_in(1, buf1, sem1)

        def body(p, carry):
            c0 = p * 2
            wait_in(c0, buf0, sem0)
            consume(c0, buf0)

            @pl.when(p + 1 < n_pairs)
            def _():
                start_in(c0 + 2, buf0, sem0)

            wait_in(c0 + 1, buf1, sem1)
            consume(c0 + 1, buf1)

            @pl.when(p + 1 < n_pairs)
            def _():
                start_in(c0 + 3, buf1, sem1)

            return carry

        lax.fori_loop(0, n_pairs, body, 0)
        pltpu.sync_copy(obuf, out_hbm.at[pl.ds(wid * rows_w * _LANES,
                                               rows_w * _LANES)])

    return k(x1, idx)


def _tc_body(y_ref, w_ref, o_ref, *, inv_n):
    pooled = jnp.sum(y_ref[...], axis=2) * inv_n
    o_ref[...] = lax.dot_general(
        pooled, w_ref[...], (((1,), (1,)), ((), ())),
        preferred_element_type=jnp.float32)


def kernel(x, W, indice):
    b, c, h, w = x.shape
    hw = h * w
    n, _ = W.shape
    rows = b * c
    x1 = x.reshape(rows * hw)
    idx = indice.astype(jnp.int32)
    n_idx = idx.shape[0]

    y = _sc_pooled(x1, idx, rows, hw, n_workers=32)
    y3 = y.reshape(b, c, _LANES)

    bb = 64
    out = pl.pallas_call(
        functools.partial(_tc_body, inv_n=1.0 / n_idx),
        grid=(b // bb,),
        in_specs=[
            pl.BlockSpec((bb, c, _LANES), lambda i: (i, 0, 0)),
            pl.BlockSpec((n, c), lambda i: (0, 0)),
        ],
        out_specs=pl.BlockSpec((bb, n), lambda i: (i, 0)),
        out_shape=jax.ShapeDtypeStruct((b, n), jnp.float32),
        compiler_params=pltpu.CompilerParams(
            dimension_semantics=("arbitrary",)),
    )(y3, W)
    return out


# speedup vs baseline: 1.0235x; 1.0235x over previous
import functools

import jax
import jax.numpy as jnp
from jax import lax
from jax.experimental import pallas as pl
from jax.experimental.pallas import tpu as pltpu
from jax.experimental.pallas import tpu_sc as plsc

_CHUNK = 16
_NBUF = 4
_LANES = 16


def _sc_pooled(x1, idx, rows, hw, n_workers):
    n_idx = idx.shape[0]
    rows_w = rows // n_workers
    n_chunks = rows_w // _CHUNK
    n_groups = n_chunks // _NBUF
    chunk_elems = _CHUNK * hw
    mesh = plsc.VectorSubcoreMesh(core_axis_name="c", subcore_axis_name="s")

    @functools.partial(
        pl.kernel, mesh=mesh,
        out_type=jax.ShapeDtypeStruct((rows * _LANES,), jnp.float32),
        scratch_types=[
            *[pltpu.VMEM((chunk_elems,), jnp.float32) for _ in range(_NBUF)],
            pltpu.VMEM((rows_w * _LANES,), jnp.float32),
            pltpu.VMEM((n_idx,), jnp.int32),
            *[pltpu.SemaphoreType.DMA for _ in range(_NBUF)],
        ],
        compiler_params=pltpu.CompilerParams(needs_layout_passes=False),
    )
    def k(x_hbm, idx_hbm, out_hbm, *refs):
        bufs = refs[:_NBUF]
        obuf, idx_v = refs[_NBUF], refs[_NBUF + 1]
        sems = refs[_NBUF + 2:]
        wid = lax.axis_index("s") * 2 + lax.axis_index("c")
        webase = wid * rows_w * hw
        pltpu.sync_copy(idx_hbm, idx_v)
        idx_regs = [idx_v[pl.ds(q * _LANES, _LANES)] for q in range(n_idx // _LANES)]

        def start_in(chunk, buf, sem):
            pltpu.make_async_copy(
                x_hbm.at[pl.ds(webase + chunk * chunk_elems, chunk_elems)],
                buf, sem).start()

        def wait_in(chunk, buf, sem):
            pltpu.make_async_copy(
                x_hbm.at[pl.ds(webase + chunk * chunk_elems, chunk_elems)],
                buf, sem).wait()

        def consume(chunk, buf):
            lbase = chunk * _CHUNK * _LANES
            for r in range(_CHUNK):
                acc = plsc.load_gather(buf, [idx_regs[0] + r * hw])
                for q in range(1, len(idx_regs)):
                    acc = acc + plsc.load_gather(buf, [idx_regs[q] + r * hw])
                obuf[pl.ds(lbase + r * _LANES, _LANES)] = acc

        for j in range(_NBUF):
            start_in(j, bufs[j], sems[j])

        def body(g, carry):
            c0 = g * _NBUF
            for j in range(_NBUF):
                wait_in(c0 + j, bufs[j], sems[j])
                consume(c0 + j, bufs[j])

                @pl.when(c0 + j + _NBUF < n_chunks)
                def _():
                    start_in(c0 + j + _NBUF, bufs[j], sems[j])

            return carry

        lax.fori_loop(0, n_groups, body, 0)
        pltpu.sync_copy(obuf, out_hbm.at[pl.ds(wid * rows_w * _LANES,
                                               rows_w * _LANES)])

    return k(x1, idx)


def _tc_body(y_ref, w_ref, o_ref, *, inv_n):
    pooled = jnp.sum(y_ref[...], axis=2) * inv_n
    o_ref[...] = lax.dot_general(
        pooled, w_ref[...], (((1,), (1,)), ((), ())),
        preferred_element_type=jnp.float32)


def kernel(x, W, indice):
    b, c, h, w = x.shape
    hw = h * w
    n, _ = W.shape
    rows = b * c
    x1 = x.reshape(rows * hw)
    idx = indice.astype(jnp.int32)
    n_idx = idx.shape[0]

    y = _sc_pooled(x1, idx, rows, hw, n_workers=32)
    y3 = y.reshape(b, c, _LANES)

    bb = 64
    out = pl.pallas_call(
        functools.partial(_tc_body, inv_n=1.0 / n_idx),
        grid=(b // bb,),
        in_specs=[
            pl.BlockSpec((bb, c, _LANES), lambda i: (i, 0, 0)),
            pl.BlockSpec((n, c), lambda i: (0, 0)),
        ],
        out_specs=pl.BlockSpec((bb, n), lambda i: (i, 0)),
        out_shape=jax.ShapeDtypeStruct((b, n), jnp.float32),
        compiler_params=pltpu.CompilerParams(
            dimension_semantics=("arbitrary",)),
    )(y3, W)
    return out


# speedup vs baseline: 1.5590x; 1.5233x over previous
import functools

import jax
import jax.numpy as jnp
from jax import lax
from jax.experimental import pallas as pl
from jax.experimental.pallas import tpu as pltpu
from jax.experimental.pallas import tpu_sc as plsc

_CHUNK = 16
_NBUF = 4
_LANES = 16


def _sc_pooled(x2, idx, n_workers):
    rows, hw = x2.shape
    n_idx = idx.shape[0]
    rows_w = rows // n_workers
    n_chunks = rows_w // _CHUNK
    n_groups = n_chunks // _NBUF
    mesh = plsc.VectorSubcoreMesh(core_axis_name="c", subcore_axis_name="s")

    @functools.partial(
        pl.kernel, mesh=mesh,
        out_type=jax.ShapeDtypeStruct((rows,), jnp.float32),
        scratch_types=[
            *[pltpu.VMEM((_CHUNK, hw), jnp.float32) for _ in range(_NBUF)],
            pltpu.VMEM((rows_w,), jnp.float32),
            pltpu.VMEM((n_idx,), jnp.int32),
            pltpu.VMEM((n_idx * _LANES,), jnp.int32),
            pltpu.VMEM((n_idx * _LANES,), jnp.int32),
            *[pltpu.SemaphoreType.DMA for _ in range(_NBUF)],
        ],
        compiler_params=pltpu.CompilerParams(needs_layout_passes=False),
    )
    def k(x_hbm, idx_hbm, out_hbm, *refs):
        bufs = refs[:_NBUF]
        obuf, idx_v, tbl_r, tbl_c = refs[_NBUF:_NBUF + 4]
        sems = refs[_NBUF + 4:]
        wid = lax.axis_index("s") * 2 + lax.axis_index("c")
        wrbase = wid * rows_w
        pltpu.sync_copy(idx_hbm, idx_v)
        lane = lax.iota(jnp.int32, _LANES)
        for q in range(n_idx // _LANES):
            v = idx_v[pl.ds(q * _LANES, _LANES)]
            for t in range(_LANES):
                pos = lane * _LANES + (q * _LANES * _LANES + t)
                plsc.store_scatter(tbl_r, [pos], jnp.full((_LANES,), t, jnp.int32))
                plsc.store_scatter(tbl_c, [pos], v)

        def start_in(chunk, buf, sem):
            pltpu.make_async_copy(
                x_hbm.at[pl.ds(wrbase + chunk * _CHUNK, _CHUNK), :],
                buf, sem).start()

        def wait_in(chunk, buf, sem):
            pltpu.make_async_copy(
                x_hbm.at[pl.ds(wrbase + chunk * _CHUNK, _CHUNK), :],
                buf, sem).wait()

        def consume(chunk, buf):
            acc = plsc.load_gather(
                buf, [tbl_r[pl.ds(0, _LANES)], tbl_c[pl.ds(0, _LANES)]])
            for s in range(1, n_idx):
                acc = acc + plsc.load_gather(
                    buf, [tbl_r[pl.ds(s * _LANES, _LANES)],
                          tbl_c[pl.ds(s * _LANES, _LANES)]])
            obuf[pl.ds(chunk * _CHUNK, _CHUNK)] = acc

        for j in range(_NBUF):
            start_in(j, bufs[j], sems[j])

        def body(g, carry):
            c0 = g * _NBUF
            for j in range(_NBUF):
                wait_in(c0 + j, bufs[j], sems[j])
                consume(c0 + j, bufs[j])

                @pl.when(c0 + j + _NBUF < n_chunks)
                def _():
                    start_in(c0 + j + _NBUF, bufs[j], sems[j])

            return carry

        lax.fori_loop(0, n_groups, body, 0)
        pltpu.sync_copy(obuf, out_hbm.at[pl.ds(wrbase, rows_w)])

    return k(x2, idx)


def _tc_body(p_ref, w_ref, o_ref, *, inv_n):
    pooled = p_ref[...] * inv_n
    o_ref[...] = lax.dot_general(
        pooled, w_ref[...], (((1,), (1,)), ((), ())),
        preferred_element_type=jnp.float32)


def kernel(x, W, indice):
    b, c, h, w = x.shape
    hw = h * w
    n, _ = W.shape
    rows = b * c
    x2 = x.reshape(rows, hw)
    idx = indice.astype(jnp.int32)
    n_idx = idx.shape[0]

    y = _sc_pooled(x2, idx, n_workers=32)
    p2 = y.reshape(b, c)

    out = pl.pallas_call(
        functools.partial(_tc_body, inv_n=1.0 / n_idx),
        in_specs=[
            pl.BlockSpec((b, c), lambda: (0, 0)),
            pl.BlockSpec((n, c), lambda: (0, 0)),
        ],
        out_specs=pl.BlockSpec((b, n), lambda: (0, 0)),
        out_shape=jax.ShapeDtypeStruct((b, n), jnp.float32),
    )(p2, W)
    return out


# speedup vs baseline: 3.3303x; 2.1361x over previous
import functools

import jax
import jax.numpy as jnp
from jax import lax
from jax.experimental import pallas as pl
from jax.experimental.pallas import tpu as pltpu
from jax.experimental.pallas import tpu_sc as plsc

_CHUNK = 16
_NBUF = 4
_LANES = 16


def _sc_pooled(x3, idx, n_workers):
    b, c, hw = x3.shape
    rows = b * c
    n_idx = idx.shape[0]
    rows_w = rows // n_workers
    n_chunks = rows_w // _CHUNK
    n_groups = n_chunks // _NBUF
    chunks_per_b = c // _CHUNK
    mesh = plsc.VectorSubcoreMesh(core_axis_name="c", subcore_axis_name="s")

    @functools.partial(
        pl.kernel, mesh=mesh,
        out_type=jax.ShapeDtypeStruct((rows,), jnp.float32),
        scratch_types=[
            *[pltpu.VMEM((_CHUNK, hw), jnp.float32) for _ in range(_NBUF)],
            pltpu.VMEM((rows_w,), jnp.float32),
            pltpu.VMEM((n_idx,), jnp.int32),
            pltpu.VMEM((n_idx * _LANES,), jnp.int32),
            pltpu.VMEM((n_idx * _LANES,), jnp.int32),
            *[pltpu.SemaphoreType.DMA for _ in range(_NBUF)],
        ],
        compiler_params=pltpu.CompilerParams(needs_layout_passes=False),
    )
    def k(x_hbm, idx_hbm, out_hbm, *refs):
        bufs = refs[:_NBUF]
        obuf, idx_v, tbl_r, tbl_c = refs[_NBUF:_NBUF + 4]
        sems = refs[_NBUF + 4:]
        wid = lax.axis_index("s") * 2 + lax.axis_index("c")
        wrbase = wid * rows_w
        pltpu.sync_copy(idx_hbm, idx_v)
        lane = lax.iota(jnp.int32, _LANES)
        for q in range(n_idx // _LANES):
            v = idx_v[pl.ds(q * _LANES, _LANES)]
            for t in range(_LANES):
                pos = lane * _LANES + (q * _LANES * _LANES + t)
                plsc.store_scatter(tbl_r, [pos], jnp.full((_LANES,), t, jnp.int32))
                plsc.store_scatter(tbl_c, [pos], v)

        def _src(chunk):
            gchunk = wid * n_chunks + chunk
            bi = gchunk // chunks_per_b
            c0 = (gchunk % chunks_per_b) * _CHUNK
            return x_hbm.at[bi, pl.ds(c0, _CHUNK), :]

        def start_in(chunk, buf, sem):
            pltpu.make_async_copy(_src(chunk), buf, sem).start()

        def wait_in(chunk, buf, sem):
            pltpu.make_async_copy(_src(chunk), buf, sem).wait()

        def consume(chunk, buf):
            acc = plsc.load_gather(
                buf, [tbl_r[pl.ds(0, _LANES)], tbl_c[pl.ds(0, _LANES)]])
            for s in range(1, n_idx):
                acc = acc + plsc.load_gather(
                    buf, [tbl_r[pl.ds(s * _LANES, _LANES)],
                          tbl_c[pl.ds(s * _LANES, _LANES)]])
            obuf[pl.ds(chunk * _CHUNK, _CHUNK)] = acc

        for j in range(_NBUF):
            start_in(j, bufs[j], sems[j])

        def body(g, carry):
            c0 = g * _NBUF
            for j in range(_NBUF):
                wait_in(c0 + j, bufs[j], sems[j])
                consume(c0 + j, bufs[j])

                @pl.when(c0 + j + _NBUF < n_chunks)
                def _():
                    start_in(c0 + j + _NBUF, bufs[j], sems[j])

            return carry

        lax.fori_loop(0, n_groups, body, 0)
        pltpu.sync_copy(obuf, out_hbm.at[pl.ds(wrbase, rows_w)])

    return k(x3, idx)


def _tc_body(p_ref, w_ref, o_ref, *, inv_n):
    pooled = p_ref[...] * inv_n
    o_ref[...] = lax.dot_general(
        pooled, w_ref[...], (((1,), (1,)), ((), ())),
        preferred_element_type=jnp.float32)


def kernel(x, W, indice):
    b, c, h, w = x.shape
    hw = h * w
    n, _ = W.shape
    rows = b * c
    x3 = x.reshape(b, c, hw)
    idx = indice.astype(jnp.int32)
    n_idx = idx.shape[0]

    y = _sc_pooled(x3, idx, n_workers=32)
    p2 = y.reshape(b, c)

    out = pl.pallas_call(
        functools.partial(_tc_body, inv_n=1.0 / n_idx),
        in_specs=[
            pl.BlockSpec((b, c), lambda: (0, 0)),
            pl.BlockSpec((n, c), lambda: (0, 0)),
        ],
        out_specs=pl.BlockSpec((b, n), lambda: (0, 0)),
        out_shape=jax.ShapeDtypeStruct((b, n), jnp.float32),
    )(p2, W)
    return out
